# baseline (speedup 1.0000x reference)
import functools

import jax
import jax.numpy as jnp
from jax.experimental import pallas as pl
from jax.experimental.pallas import tpu as pltpu


def _resize_body(fov0_ref, w0_ref, fov1_ref, w1_ref, xa_ref, xb_ref, o_ref,
                 a0t_ref, a1t_ref):
    taps = fov0_ref.shape[0]

    @pl.when(pl.program_id(0) == 0)
    def _densify():
        for fov_ref, w_ref, out_ref in ((fov0_ref, w0_ref, a0t_ref),
                                        (fov1_ref, w1_ref, a1t_ref)):
            in_len, out_len = out_ref.shape
            row = jax.lax.broadcasted_iota(jnp.int32, (in_len, out_len), 0)
            acc = jnp.zeros((in_len, out_len), jnp.float32)
            for p in range(taps):
                acc += jnp.where(fov_ref[p : p + 1, :] == row,
                                 w_ref[p : p + 1, :], 0.0)
            out_ref[...] = acc

    half = xa_ref.shape[0]
    for j in range(2 * half):
        x_j = xa_ref[j] if j < half else xb_ref[j - half]
        y = jax.lax.dot_general(
            a0t_ref[...], x_j,
            dimension_numbers=(((0,), (0,)), ((), ())),
            preferred_element_type=jnp.float32,
        )
        o_ref[j] = jnp.dot(y, a1t_ref[...], preferred_element_type=jnp.float32)


@functools.partial(jax.jit, static_argnames=("block", "interpret"))
def _resize(fov0, w0, fov1, w1, x, block=4, interpret=False):
    n, h, w = x.shape
    taps, oh = fov0.shape
    ow = fov1.shape[1]
    return pl.pallas_call(
        _resize_body,
        grid=(n // block,),
        in_specs=[
            pl.BlockSpec((taps, oh), lambda i: (0, 0)),
            pl.BlockSpec((taps, oh), lambda i: (0, 0)),
            pl.BlockSpec((taps, ow), lambda i: (0, 0)),
            pl.BlockSpec((taps, ow), lambda i: (0, 0)),
            pl.BlockSpec((block // 2, h, w), lambda i: (2 * i, 0, 0)),
            pl.BlockSpec((block // 2, h, w), lambda i: (2 * i + 1, 0, 0)),
        ],
        out_specs=pl.BlockSpec((block, oh, ow), lambda i: (i, 0, 0)),
        out_shape=jax.ShapeDtypeStruct((n, oh, ow), jnp.float32),
        scratch_shapes=[
            pltpu.VMEM((h, oh), jnp.float32),
            pltpu.VMEM((w, ow), jnp.float32),
        ],
        compiler_params=pltpu.CompilerParams(
            dimension_semantics=("arbitrary",),
        ),
        interpret=interpret,
    )(fov0, w0, fov1, w1, x, x)


def kernel(in_tensor, w0, w1, fov0, fov1, interpret=False):
    b, c, h, w = in_tensor.shape
    taps, oh = fov0.shape
    ow = fov1.shape[1]
    x = in_tensor.reshape(b * c, h, w)
    out = _resize(fov0.astype(jnp.int32),
                  w0.reshape(taps, oh).astype(jnp.float32),
                  fov1.astype(jnp.int32),
                  w1.reshape(taps, ow).astype(jnp.float32),
                  x, block=4, interpret=interpret)
    return out.reshape(b, c, oh, ow)


# speedup vs baseline: 1.0286x; 1.0286x over previous
import functools

import jax
import jax.numpy as jnp
from jax.experimental import pallas as pl
from jax.experimental.pallas import tpu as pltpu


def _resize_body(fov0_ref, w0_ref, fov1_ref, w1_ref, x_ref, o_ref,
                 a0t_ref, a1t_ref):
    taps = fov0_ref.shape[0]

    @pl.when(pl.program_id(0) == 0)
    def _densify():
        for fov_ref, w_ref, out_ref in ((fov0_ref, w0_ref, a0t_ref),
                                        (fov1_ref, w1_ref, a1t_ref)):
            in_len, out_len = out_ref.shape
            row = jax.lax.broadcasted_iota(jnp.int32, (in_len, out_len), 0)
            acc = jnp.zeros((in_len, out_len), jnp.float32)
            for p in range(taps):
                acc += jnp.where(fov_ref[p : p + 1, :] == row,
                                 w_ref[p : p + 1, :], 0.0)
            out_ref[...] = acc

    nb = x_ref.shape[0]
    for j in range(nb):
        y = jax.lax.dot_general(
            a0t_ref[...], x_ref[j],
            dimension_numbers=(((0,), (0,)), ((), ())),
            preferred_element_type=jnp.float32,
        )
        o_ref[j] = jnp.dot(y, a1t_ref[...], preferred_element_type=jnp.float32)


@functools.partial(jax.jit, static_argnames=("block", "interpret"))
def _resize(fov0, w0, fov1, w1, x, block=4, interpret=False):
    n, h, w = x.shape
    taps, oh = fov0.shape
    ow = fov1.shape[1]
    return pl.pallas_call(
        _resize_body,
        grid=(n // block,),
        in_specs=[
            pl.BlockSpec((taps, oh), lambda i: (0, 0)),
            pl.BlockSpec((taps, oh), lambda i: (0, 0)),
            pl.BlockSpec((taps, ow), lambda i: (0, 0)),
            pl.BlockSpec((taps, ow), lambda i: (0, 0)),
            pl.BlockSpec((block, h, w), lambda i: (i, 0, 0)),
        ],
        out_specs=pl.BlockSpec((block, oh, ow), lambda i: (i, 0, 0)),
        out_shape=jax.ShapeDtypeStruct((n, oh, ow), jnp.float32),
        scratch_shapes=[
            pltpu.VMEM((h, oh), jnp.float32),
            pltpu.VMEM((w, ow), jnp.float32),
        ],
        compiler_params=pltpu.CompilerParams(
            dimension_semantics=("arbitrary",),
        ),
        interpret=interpret,
    )(fov0, w0, fov1, w1, x)


def kernel(in_tensor, w0, w1, fov0, fov1, interpret=False):
    b, c, h, w = in_tensor.shape
    taps, oh = fov0.shape
    ow = fov1.shape[1]
    x = in_tensor.reshape(b * c, h, w)
    out = _resize(fov0.astype(jnp.int32),
                  w0.reshape(taps, oh).astype(jnp.float32),
                  fov1.astype(jnp.int32),
                  w1.reshape(taps, ow).astype(jnp.float32),
                  x, block=4, interpret=interpret)
    return out.reshape(b, c, oh, ow)
